# baseline (device time: 26513 ns/iter reference)
import jax
import jax.numpy as jnp
from jax import lax
from jax.experimental import pallas as pl
from jax.experimental.pallas import tpu as pltpu

C = 8


def kernel(partial, resid, gamma):
    m, d = resid.shape
    half = m // 2
    rows = half // C

    def body(part_ref, resid_ref, gamma_ref, out_ref,
             part_v, resid_v, gamma_v, mine, other_half, out_bf, recv_out,
             stage_my, stage_ot,
             sa, ra, sb, rb, sem_part, sem_resid, sem_gamma,
             sem_out_my, sem_out_ot):
        my_x = lax.axis_index("x")
        my_y = lax.axis_index("y")
        x_nbr = (1 - my_x, my_y)
        y_nbr = (my_x, 1 - my_y)

        barrier = pltpu.get_barrier_semaphore()
        for nbr in (x_nbr, y_nbr):
            pl.semaphore_signal(barrier, inc=1, device_id=nbr,
                                device_id_type=pl.DeviceIdType.MESH)
        pl.semaphore_wait(barrier, 2)

        row0 = my_y * half
        orow0 = (1 - my_y) * half

        gamma_cp = pltpu.make_async_copy(gamma_ref, gamma_v, sem_gamma)
        gamma_cp.start()
        resid_cp = pltpu.make_async_copy(
            resid_ref.at[pl.ds(row0, half)], resid_v, sem_resid)
        resid_cp.start()
        part_cps = []
        for c in range(C):
            lo = c * rows
            cp = pltpu.make_async_copy(
                part_ref.at[0, pl.ds(row0 + lo, rows)],
                part_v.at[pl.ds(lo, rows)],
                sem_part.at[c])
            cp.start()
            part_cps.append(cp)

        a_rdmas = []
        for c in range(C):
            lo = c * rows
            part_cps[c].wait()
            mine[pl.ds(lo, rows), :] = (
                part_v[pl.ds(lo, rows), :].astype(jnp.bfloat16))
            r = pltpu.make_async_remote_copy(
                src_ref=mine.at[pl.ds(lo, rows)],
                dst_ref=other_half.at[pl.ds(lo, rows)],
                send_sem=sa.at[c], recv_sem=ra.at[c],
                device_id=x_nbr, device_id_type=pl.DeviceIdType.MESH,
            )
            r.start()
            a_rdmas.append(r)
        gamma_cp.wait()
        resid_cp.wait()

        b_rdmas, out_my_cps = [], []
        for c in range(C):
            lo = c * rows
            a_rdmas[c].wait_recv()
            yv = (part_v[pl.ds(lo, rows), :]
                  + other_half[pl.ds(lo, rows), :].astype(jnp.float32)
                  + resid_v[pl.ds(lo, rows), :])
            ms = jnp.mean(yv * yv, axis=-1, keepdims=True)
            scaled = yv * lax.rsqrt(ms + 1e-6) * gamma_v[...]
            stage_my[pl.ds(lo, rows), :] = scaled
            out_bf[pl.ds(lo, rows), :] = scaled.astype(jnp.bfloat16)
            r = pltpu.make_async_remote_copy(
                src_ref=out_bf.at[pl.ds(lo, rows)],
                dst_ref=recv_out.at[pl.ds(lo, rows)],
                send_sem=sb.at[c], recv_sem=rb.at[c],
                device_id=y_nbr, device_id_type=pl.DeviceIdType.MESH,
            )
            r.start()
            b_rdmas.append(r)
            cp = pltpu.make_async_copy(
                stage_my.at[pl.ds(lo, rows)],
                out_ref.at[pl.ds(row0 + lo, rows)],
                sem_out_my.at[c])
            cp.start()
            out_my_cps.append(cp)

        out_ot_cps = []
        for c in range(C):
            lo = c * rows
            b_rdmas[c].wait_recv()
            stage_ot[pl.ds(lo, rows), :] = (
                recv_out[pl.ds(lo, rows), :].astype(jnp.float32))
            cp = pltpu.make_async_copy(
                stage_ot.at[pl.ds(lo, rows)],
                out_ref.at[pl.ds(orow0 + lo, rows)],
                sem_out_ot.at[c])
            cp.start()
            out_ot_cps.append(cp)

        for c in range(C):
            a_rdmas[c].wait_send()
            b_rdmas[c].wait_send()
            out_my_cps[c].wait()
            out_ot_cps[c].wait()

    return pl.pallas_call(
        body,
        out_shape=jax.ShapeDtypeStruct((m, d), jnp.float32),
        in_specs=[pl.BlockSpec(memory_space=pl.MemorySpace.ANY)] * 3,
        out_specs=pl.BlockSpec(memory_space=pl.MemorySpace.ANY),
        scratch_shapes=[
            pltpu.VMEM((half, d), jnp.float32),
            pltpu.VMEM((half, d), jnp.float32),
            pltpu.VMEM((1, d), jnp.float32),
            pltpu.VMEM((half, d), jnp.bfloat16),
            pltpu.VMEM((half, d), jnp.bfloat16),
            pltpu.VMEM((half, d), jnp.bfloat16),
            pltpu.VMEM((half, d), jnp.bfloat16),
            pltpu.VMEM((half, d), jnp.float32),
            pltpu.VMEM((half, d), jnp.float32),
            pltpu.SemaphoreType.DMA((C,)),
            pltpu.SemaphoreType.DMA((C,)),
            pltpu.SemaphoreType.DMA((C,)),
            pltpu.SemaphoreType.DMA((C,)),
            pltpu.SemaphoreType.DMA((C,)),
            pltpu.SemaphoreType.DMA,
            pltpu.SemaphoreType.DMA,
            pltpu.SemaphoreType.DMA((C,)),
            pltpu.SemaphoreType.DMA((C,)),
        ],
        compiler_params=pltpu.CompilerParams(collective_id=0),
    )(partial, resid, gamma.reshape(1, d))


# device time: 22375 ns/iter; 1.1849x vs baseline; 1.1849x over previous
import jax
import jax.numpy as jnp
from jax import lax
from jax.experimental import pallas as pl
from jax.experimental.pallas import tpu as pltpu

C = 8


def kernel(partial, resid, gamma):
    m, d = resid.shape
    half = m // 2
    rows = half // C

    def body(part_ref, resid_ref, gamma_ref, out_ref,
             part_v, resid_v, gamma_v, mine, other_half, out_bf, recv_out,
             stage_my, stage_ot,
             sa, ra, sb, rb, sem_part, sem_resid, sem_gamma,
             sem_out_my, sem_out_ot):
        my_x = lax.axis_index("x")
        my_y = lax.axis_index("y")
        x_nbr = (1 - my_x, my_y)
        y_nbr = (my_x, 1 - my_y)

        barrier = pltpu.get_barrier_semaphore()
        for nbr in (x_nbr, y_nbr):
            pl.semaphore_signal(barrier, inc=1, device_id=nbr,
                                device_id_type=pl.DeviceIdType.MESH)
        pl.semaphore_wait(barrier, 2)

        row0 = my_y * half
        orow0 = (1 - my_y) * half

        gamma_cp = pltpu.make_async_copy(gamma_ref, gamma_v, sem_gamma)
        gamma_cp.start()
        resid_cp = pltpu.make_async_copy(
            resid_ref.at[pl.ds(row0, half)], resid_v, sem_resid)
        resid_cp.start()
        part_cps = []
        for c in range(C):
            lo = c * rows
            cp = pltpu.make_async_copy(
                part_ref.at[0, pl.ds(row0 + lo, rows)],
                part_v.at[pl.ds(lo, rows)],
                sem_part.at[c])
            cp.start()
            part_cps.append(cp)

        a_rdmas = []
        for c in range(C):
            lo = c * rows
            part_cps[c].wait()
            mine[pl.ds(lo, rows), :] = (
                part_v[pl.ds(lo, rows), :].astype(jnp.bfloat16))
            r = pltpu.make_async_remote_copy(
                src_ref=mine.at[pl.ds(lo, rows)],
                dst_ref=other_half.at[pl.ds(lo, rows)],
                send_sem=sa.at[c], recv_sem=ra.at[c],
                device_id=x_nbr, device_id_type=pl.DeviceIdType.MESH,
            )
            r.start()
            a_rdmas.append(r)
        gamma_cp.wait()
        resid_cp.wait()

        b_rdmas, out_my_cps = [], []
        for c in range(C):
            lo = c * rows
            a_rdmas[c].wait_recv()
            yv = (part_v[pl.ds(lo, rows), :]
                  + other_half[pl.ds(lo, rows), :].astype(jnp.float32)
                  + resid_v[pl.ds(lo, rows), :])
            ms = jnp.mean(yv * yv, axis=-1, keepdims=True)
            scaled = yv * lax.rsqrt(ms + 1e-6) * gamma_v[...]
            stage_my[pl.ds(lo, rows), :] = scaled
            out_bf[pl.ds(lo, rows), :] = scaled.astype(jnp.bfloat16)
            r = pltpu.make_async_remote_copy(
                src_ref=out_bf.at[pl.ds(lo, rows)],
                dst_ref=recv_out.at[pl.ds(lo, rows)],
                send_sem=sb.at[c], recv_sem=rb.at[c],
                device_id=y_nbr, device_id_type=pl.DeviceIdType.MESH,
            )
            r.start()
            b_rdmas.append(r)
            cp = pltpu.make_async_copy(
                stage_my.at[pl.ds(lo, rows)],
                out_ref.at[pl.ds(row0 + lo, rows)],
                sem_out_my.at[c])
            cp.start()
            out_my_cps.append(cp)

        out_ot_cps = []
        for c in range(C):
            lo = c * rows
            b_rdmas[c].wait_recv()
            stage_ot[pl.ds(lo, rows), :] = (
                recv_out[pl.ds(lo, rows), :].astype(jnp.float32))
            cp = pltpu.make_async_copy(
                stage_ot.at[pl.ds(lo, rows)],
                out_ref.at[pl.ds(orow0 + lo, rows)],
                sem_out_ot.at[c])
            cp.start()
            out_ot_cps.append(cp)

        for c in range(C):
            a_rdmas[c].wait_send()
            b_rdmas[c].wait_send()
            out_my_cps[c].wait()
            out_ot_cps[c].wait()

    return pl.pallas_call(
        body,
        out_shape=jax.ShapeDtypeStruct((m, d), jnp.float32),
        in_specs=[pl.BlockSpec(memory_space=pltpu.MemorySpace.HBM)] * 3,
        out_specs=pl.BlockSpec(memory_space=pltpu.MemorySpace.HBM),
        scratch_shapes=[
            pltpu.VMEM((half, d), jnp.float32),
            pltpu.VMEM((half, d), jnp.float32),
            pltpu.VMEM((1, d), jnp.float32),
            pltpu.VMEM((half, d), jnp.bfloat16),
            pltpu.VMEM((half, d), jnp.bfloat16),
            pltpu.VMEM((half, d), jnp.bfloat16),
            pltpu.VMEM((half, d), jnp.bfloat16),
            pltpu.VMEM((half, d), jnp.float32),
            pltpu.VMEM((half, d), jnp.float32),
            pltpu.SemaphoreType.DMA((C,)),
            pltpu.SemaphoreType.DMA((C,)),
            pltpu.SemaphoreType.DMA((C,)),
            pltpu.SemaphoreType.DMA((C,)),
            pltpu.SemaphoreType.DMA((C,)),
            pltpu.SemaphoreType.DMA,
            pltpu.SemaphoreType.DMA,
            pltpu.SemaphoreType.DMA((C,)),
            pltpu.SemaphoreType.DMA((C,)),
        ],
        compiler_params=pltpu.CompilerParams(collective_id=0),
    )(
        pltpu.with_memory_space_constraint(partial, pltpu.MemorySpace.HBM),
        pltpu.with_memory_space_constraint(resid, pltpu.MemorySpace.HBM),
        pltpu.with_memory_space_constraint(
            gamma.reshape(1, d), pltpu.MemorySpace.HBM),
    )


# device time: 21759 ns/iter; 1.2185x vs baseline; 1.0283x over previous
import jax
import jax.numpy as jnp
from jax import lax
from jax.experimental import pallas as pl
from jax.experimental.pallas import tpu as pltpu

C = 16


def kernel(partial, resid, gamma):
    m, d = resid.shape
    half = m // 2
    rows = half // C

    def body(part_ref, resid_ref, gamma_ref, out_ref,
             part_v, resid_v, gamma_v, mine, other_half, out_bf, recv_out,
             sa, ra, sb, rb, sem_part, sem_resid, sem_gamma):
        my_x = lax.axis_index("x")
        my_y = lax.axis_index("y")
        x_nbr = (1 - my_x, my_y)
        y_nbr = (my_x, 1 - my_y)

        barrier = pltpu.get_barrier_semaphore()
        for nbr in (x_nbr, y_nbr):
            pl.semaphore_signal(barrier, inc=1, device_id=nbr,
                                device_id_type=pl.DeviceIdType.MESH)
        pl.semaphore_wait(barrier, 2)

        row0 = my_y * half
        orow0 = (1 - my_y) * half

        gamma_cp = pltpu.make_async_copy(gamma_ref, gamma_v, sem_gamma)
        gamma_cp.start()
        resid_cp = pltpu.make_async_copy(
            resid_ref.at[pl.ds(row0, half)], resid_v, sem_resid)
        resid_cp.start()
        part_cps = []
        for c in range(C):
            lo = c * rows
            cp = pltpu.make_async_copy(
                part_ref.at[0, pl.ds(row0 + lo, rows)],
                part_v.at[pl.ds(lo, rows)],
                sem_part.at[c])
            cp.start()
            part_cps.append(cp)

        a_rdmas = []
        for c in range(C):
            lo = c * rows
            part_cps[c].wait()
            mine[pl.ds(lo, rows), :] = (
                part_v[pl.ds(lo, rows), :].astype(jnp.bfloat16))
            r = pltpu.make_async_remote_copy(
                src_ref=mine.at[pl.ds(lo, rows)],
                dst_ref=other_half.at[pl.ds(lo, rows)],
                send_sem=sa.at[c], recv_sem=ra.at[c],
                device_id=x_nbr, device_id_type=pl.DeviceIdType.MESH,
            )
            r.start()
            a_rdmas.append(r)
        gamma_cp.wait()
        resid_cp.wait()

        b_rdmas = []
        for c in range(C):
            lo = c * rows
            a_rdmas[c].wait_recv()
            yv = (part_v[pl.ds(lo, rows), :]
                  + other_half[pl.ds(lo, rows), :].astype(jnp.float32)
                  + resid_v[pl.ds(lo, rows), :])
            ms = jnp.mean(yv * yv, axis=-1, keepdims=True)
            scaled = yv * lax.rsqrt(ms + 1e-6) * gamma_v[...]
            out_ref[pl.ds(row0 + lo, rows), :] = scaled
            out_bf[pl.ds(lo, rows), :] = scaled.astype(jnp.bfloat16)
            r = pltpu.make_async_remote_copy(
                src_ref=out_bf.at[pl.ds(lo, rows)],
                dst_ref=recv_out.at[pl.ds(lo, rows)],
                send_sem=sb.at[c], recv_sem=rb.at[c],
                device_id=y_nbr, device_id_type=pl.DeviceIdType.MESH,
            )
            r.start()
            b_rdmas.append(r)

        for c in range(C):
            lo = c * rows
            b_rdmas[c].wait_recv()
            out_ref[pl.ds(orow0 + lo, rows), :] = (
                recv_out[pl.ds(lo, rows), :].astype(jnp.float32))

        for c in range(C):
            a_rdmas[c].wait_send()
            b_rdmas[c].wait_send()

    return pl.pallas_call(
        body,
        out_shape=jax.ShapeDtypeStruct((m, d), jnp.float32),
        in_specs=[pl.BlockSpec(memory_space=pltpu.MemorySpace.HBM)] * 3,
        out_specs=pl.BlockSpec(memory_space=pltpu.VMEM),
        scratch_shapes=[
            pltpu.VMEM((half, d), jnp.float32),
            pltpu.VMEM((half, d), jnp.float32),
            pltpu.VMEM((1, d), jnp.float32),
            pltpu.VMEM((half, d), jnp.bfloat16),
            pltpu.VMEM((half, d), jnp.bfloat16),
            pltpu.VMEM((half, d), jnp.bfloat16),
            pltpu.VMEM((half, d), jnp.bfloat16),
            pltpu.SemaphoreType.DMA((C,)),
            pltpu.SemaphoreType.DMA((C,)),
            pltpu.SemaphoreType.DMA((C,)),
            pltpu.SemaphoreType.DMA((C,)),
            pltpu.SemaphoreType.DMA((C,)),
            pltpu.SemaphoreType.DMA,
            pltpu.SemaphoreType.DMA,
        ],
        compiler_params=pltpu.CompilerParams(collective_id=0),
    )(
        pltpu.with_memory_space_constraint(partial, pltpu.MemorySpace.HBM),
        pltpu.with_memory_space_constraint(resid, pltpu.MemorySpace.HBM),
        pltpu.with_memory_space_constraint(
            gamma.reshape(1, d), pltpu.MemorySpace.HBM),
    )
